# baseline (speedup 1.0000x reference)
import functools

import jax
import jax.numpy as jnp
from jax import lax
from jax.experimental import pallas as pl
from jax.experimental.pallas import tpu as pltpu
from jax.experimental.pallas import tpu_sc as plsc

BATCH = 1024
SEQ = 512
FEAT = 64
LANES = 16
CHUNK = 128
NBUF = 4
FT = FEAT // 8
TILE = 8 * CHUNK


def kernel(tokens, node_span_starts, node_span_ends, embed_table, pos_emb):
    del node_span_starts, node_span_ends
    num_cores, num_subcores = 2, 16
    num_workers = num_cores * num_subcores
    rows_per_w = BATCH // num_workers
    slab = rows_per_w * SEQ

    mesh = plsc.VectorSubcoreMesh(core_axis_name="c", subcore_axis_name="s")

    @functools.partial(
        pl.kernel,
        mesh=mesh,
        compiler_params=pltpu.CompilerParams(use_tc_tiling_on_sc=False,
                                             needs_layout_passes=False),
        out_type=jax.ShapeDtypeStruct((BATCH, FT, SEQ // CHUNK, 8, CHUNK),
                                      jnp.float32),
        scratch_types=[
            pltpu.VMEM((slab,), jnp.int32),
            pltpu.VMEM((SEQ * FEAT,), jnp.float32),
            [pltpu.VMEM((CHUNK, FEAT), jnp.float32) for _ in range(NBUF)],
            [pltpu.VMEM((FT, 1, 8, CHUNK), jnp.float32) for _ in range(NBUF)],
            [pltpu.SemaphoreType.DMA for _ in range(NBUF)],
            [pltpu.SemaphoreType.DMA for _ in range(NBUF)],
        ],
    )
    def body(tok_hbm, table_hbm, pos_hbm, out_hbm, tok_v, pos_v, gbufs, tbufs,
             gsems, wsems):
        wid = lax.axis_index("s") * num_cores + lax.axis_index("c")
        base = wid * rows_per_w
        pltpu.sync_copy(tok_hbm.at[pl.ds(base * SEQ, slab)], tok_v)
        pltpu.sync_copy(pos_hbm.at[pl.ds(0, SEQ * FEAT)], pos_v)

        lane = jax.lax.iota(jnp.int32, LANES)
        half = jax.lax.shift_right_logical(lane, 3)
        frow = [half + jnp.int32(2 * j) for j in range(FEAT // LANES)]
        fmid = jax.lax.bitwise_and(lane, jnp.int32(7))
        zero = lane - lane

        def fire_gather(b, r):
            idx = tok_v.at[pl.ds(r * SEQ + b * CHUNK, CHUNK)]
            pltpu.async_copy(table_hbm.at[idx], gbufs[b], gsems[b])

        def wait_gather(b):
            pltpu.make_async_copy(table_hbm.at[pl.ds(0, CHUNK)], gbufs[b],
                                  gsems[b]).wait()

        def fire_write(b, r):
            dst = out_hbm.at[base + r, pl.ds(0, FT), pl.ds(b, 1), pl.ds(0, 8),
                             pl.ds(0, CHUNK)]
            pltpu.async_copy(tbufs[b], dst, wsems[b])

        def wait_write(b):
            pltpu.make_async_copy(tbufs[b],
                                  out_hbm.at[0, pl.ds(0, FT), pl.ds(0, 1),
                                             pl.ds(0, 8), pl.ds(0, CHUNK)],
                                  wsems[b]).wait()

        def add_chunk(b):
            @plsc.parallel_loop(0, CHUNK, unroll=8)
            def _(i):
                col = jax.lax.broadcast(i, (LANES,))
                for j in range(FEAT // LANES):
                    v = (gbufs[b][i, pl.ds(j * LANES, LANES)]
                         + pos_v[pl.ds((b * CHUNK + i) * FEAT + j * LANES,
                                       LANES)])
                    plsc.store_scatter(tbufs[b], [frow[j], zero, fmid, col], v)

        for b in range(NBUF):
            fire_gather(b, 0)
        for b in range(NBUF):
            wait_gather(b)
            add_chunk(b)
            fire_write(b, 0)
            fire_gather(b, 1)

        def row_body(r, carry):
            for b in range(NBUF):
                wait_gather(b)
                wait_write(b)
                add_chunk(b)
                fire_write(b, r)
                fire_gather(b, r + 1)
            return carry

        lax.fori_loop(1, rows_per_w - 1, row_body, 0)

        last = rows_per_w - 1
        for b in range(NBUF):
            wait_gather(b)
            wait_write(b)
            add_chunk(b)
            fire_write(b, last)
        for b in range(NBUF):
            wait_write(b)

    out = body(tokens.reshape(-1), embed_table, pos_emb.reshape(-1))
    out = out.transpose(0, 2, 4, 1, 3)
    return out.reshape(BATCH, SEQ, FEAT)


# speedup vs baseline: 3.1169x; 3.1169x over previous
import functools

import jax
import jax.numpy as jnp
from jax import lax
from jax.experimental import pallas as pl
from jax.experimental.pallas import tpu as pltpu
from jax.experimental.pallas import tpu_sc as plsc

BATCH = 1024
SEQ = 512
FEAT = 64
LANES = 16
CHUNK = 128
NBUF = 4
FT = FEAT // 8
TILE = 8 * CHUNK


def kernel(tokens, node_span_starts, node_span_ends, embed_table, pos_emb):
    del node_span_starts, node_span_ends
    num_cores, num_subcores = 2, 16
    num_workers = num_cores * num_subcores
    rows_per_w = BATCH // num_workers
    slab = rows_per_w * SEQ

    mesh = plsc.VectorSubcoreMesh(core_axis_name="c", subcore_axis_name="s")

    @functools.partial(
        pl.kernel,
        mesh=mesh,
        compiler_params=pltpu.CompilerParams(use_tc_tiling_on_sc=False,
                                             needs_layout_passes=False),
        out_type=jax.ShapeDtypeStruct((BATCH, FT, SEQ // CHUNK, 8, CHUNK),
                                      jnp.float32),
        scratch_types=[
            pltpu.VMEM((slab,), jnp.int32),
            pltpu.VMEM((SEQ * FEAT,), jnp.float32),
            [pltpu.VMEM((CHUNK, FEAT), jnp.float32) for _ in range(NBUF)],
            [pltpu.VMEM((FT, 1, 8, CHUNK + 1), jnp.float32) for _ in range(NBUF)],
            [pltpu.SemaphoreType.DMA for _ in range(NBUF)],
            [pltpu.SemaphoreType.DMA for _ in range(NBUF)],
        ],
    )
    def body(tok_hbm, table_hbm, pos_hbm, out_hbm, tok_v, pos_v, gbufs, tbufs,
             gsems, wsems):
        wid = lax.axis_index("s") * num_cores + lax.axis_index("c")
        base = wid * rows_per_w
        pltpu.sync_copy(tok_hbm.at[pl.ds(base * SEQ, slab)], tok_v)
        pltpu.sync_copy(pos_hbm.at[pl.ds(0, SEQ * FEAT)], pos_v)

        lane = jax.lax.iota(jnp.int32, LANES)
        half = jax.lax.shift_right_logical(lane, 3)
        frow = [half + jnp.int32(2 * j) for j in range(FEAT // LANES)]
        fmid = jax.lax.bitwise_and(lane, jnp.int32(7))
        zero = lane - lane

        def fire_gather(b, r):
            idx = tok_v.at[pl.ds(r * SEQ + b * CHUNK, CHUNK)]
            pltpu.async_copy(table_hbm.at[idx], gbufs[b], gsems[b])

        def wait_gather(b):
            pltpu.make_async_copy(table_hbm.at[pl.ds(0, CHUNK)], gbufs[b],
                                  gsems[b]).wait()

        def fire_write(b, r):
            src = tbufs[b].at[pl.ds(0, FT), pl.ds(0, 1), pl.ds(0, 8),
                              pl.ds(0, CHUNK)]
            dst = out_hbm.at[base + r, pl.ds(0, FT), pl.ds(b, 1), pl.ds(0, 8),
                             pl.ds(0, CHUNK)]
            pltpu.async_copy(src, dst, wsems[b])

        def wait_write(b):
            src = tbufs[b].at[pl.ds(0, FT), pl.ds(0, 1), pl.ds(0, 8),
                              pl.ds(0, CHUNK)]
            pltpu.make_async_copy(src,
                                  out_hbm.at[0, pl.ds(0, FT), pl.ds(0, 1),
                                             pl.ds(0, 8), pl.ds(0, CHUNK)],
                                  wsems[b]).wait()

        def add_chunk(b):
            @plsc.parallel_loop(0, CHUNK, unroll=8)
            def _(i):
                col = jax.lax.broadcast(i, (LANES,))
                for j in range(FEAT // LANES):
                    v = (gbufs[b][i, pl.ds(j * LANES, LANES)]
                         + pos_v[pl.ds((b * CHUNK + i) * FEAT + j * LANES,
                                       LANES)])
                    plsc.store_scatter(tbufs[b], [frow[j], zero, fmid, col], v)

        for b in range(NBUF):
            fire_gather(b, 0)
        for b in range(NBUF):
            wait_gather(b)
            add_chunk(b)
            fire_write(b, 0)
            fire_gather(b, 1)

        def row_body(r, carry):
            for b in range(NBUF):
                wait_gather(b)
                wait_write(b)
                add_chunk(b)
                fire_write(b, r)
                fire_gather(b, r + 1)
            return carry

        lax.fori_loop(1, rows_per_w - 1, row_body, 0)

        last = rows_per_w - 1
        for b in range(NBUF):
            wait_gather(b)
            wait_write(b)
            add_chunk(b)
            fire_write(b, last)
        for b in range(NBUF):
            wait_write(b)

    out = body(tokens.reshape(-1), embed_table, pos_emb.reshape(-1))
    out = out.transpose(0, 2, 4, 1, 3)
    return out.reshape(BATCH, SEQ, FEAT)
